# baseline (speedup 1.0000x reference)
import dataclasses
import functools

import jax
import jax.numpy as jnp
from jax import lax
from jax.experimental import pallas as pl
from jax.experimental.pallas import tpu as pltpu
from jax.experimental.pallas import tpu_sc as plsc

N = 50000
M = 25000
C = 128
E = 400000
K = 27
G = 8
KN = K * N

BN = 400
BM = 1000
EW = 128
NWINP = 3136
EP = NWINP * EW
BW = 112
EB = BW * EW
EW2 = 64
NWIN2 = EP // EW2
TSUB = NWIN2 // 16
MH = 12800
AR = MH + 64
ASUB = AR // 16
OSUB = MH // 16
DW = 40
NDWIN = M // DW
KH = 14
NB = 2
NBE = 3
_EPS = 1e-5

_SC_MESH = dict(core_axis_name="c", subcore_axis_name="s")


def _sc_params():
    cp = pltpu.CompilerParams()
    return dataclasses.replace(cp, needs_layout_passes=False,
                               use_tc_tiling_on_sc=False)


def _gn_tc(h, gamma, beta, groups):
    n, c = h.shape
    cs = c // groups
    gi = lax.broadcasted_iota(jnp.int32, (c, groups), 0) // cs
    gj = lax.broadcasted_iota(jnp.int32, (c, groups), 1)
    Gm = (gi == gj).astype(jnp.float32)
    ti = lax.broadcasted_iota(jnp.int32, (groups, c), 0)
    tj = lax.broadcasted_iota(jnp.int32, (groups, c), 1) // cs
    GmT = (ti == tj).astype(jnp.float32)
    inv_cs = 1.0 / cs
    mu = jnp.dot(h, Gm, preferred_element_type=jnp.float32) * inv_cs
    ex2 = jnp.dot(h * h, Gm, preferred_element_type=jnp.float32) * inv_cs
    var = ex2 - mu * mu
    rstd = lax.rsqrt(var + _EPS)
    mub = jnp.dot(mu, GmT, preferred_element_type=jnp.float32)
    rstdb = jnp.dot(rstd, GmT, preferred_element_type=jnp.float32)
    return (h - mub) * rstdb * gamma + beta


def _stage_a_body(nk, x_ref, w1_ref, g1_ref, b1_ref, w2_ref, xw_ref):
    h = jnp.dot(x_ref[...], w1_ref[...], preferred_element_type=jnp.float32)
    h = _gn_tc(h, g1_ref[...], b1_ref[...], G)
    h = jnp.where(h >= 0, h, 0.01 * h)
    for k in range(nk):
        xw_ref[k] = jnp.dot(h, w2_ref[k], preferred_element_type=jnp.float32)


def _stage_a(x, W1, g1, b1, W2half):
    nk = W2half.shape[0]
    return pl.pallas_call(
        functools.partial(_stage_a_body, nk),
        grid=(N // BN,),
        in_specs=[
            pl.BlockSpec((BN, C), lambda i: (i, 0)),
            pl.BlockSpec((C, C), lambda i: (0, 0)),
            pl.BlockSpec((1, C), lambda i: (0, 0)),
            pl.BlockSpec((1, C), lambda i: (0, 0)),
            pl.BlockSpec((nk, C, C), lambda i: (0, 0, 0)),
        ],
        out_specs=pl.BlockSpec((nk, BN, C), lambda i: (0, i, 0)),
        out_shape=jax.ShapeDtypeStruct((nk, N, C), jnp.float32),
    )(x, W1, g1.reshape(1, C), b1.reshape(1, C), W2half)


def _tri_lanes(n):
    a = lax.broadcasted_iota(jnp.int32, (n, n), 0)
    b = lax.broadcasted_iota(jnp.int32, (n, n), 1)
    return (a < b).astype(jnp.float32)


def _tri_rows(n):
    a = lax.broadcasted_iota(jnp.int32, (n, n), 0)
    b = lax.broadcasted_iota(jnp.int32, (n, n), 1)
    return (b < a).astype(jnp.float32)


def _stage_p_body(ii_ref, kf_ref, oi_ref, p_ref, cnt_ref, carry):
    i = pl.program_id(0)

    @pl.when(i == 0)
    def _():
        for k in range(4):
            carry[k] = 0

    ii = ii_ref[...]
    kf = kf_ref[...]
    oi = oi_ref[...]
    row = lax.broadcasted_iota(jnp.int32, (BW, EW), 0)
    lane = lax.broadcasted_iota(jnp.int32, (BW, EW), 1)
    eidx = i * EB + row * EW + lane
    is_pad = eidx >= E
    kf_loc = jnp.where(kf >= KH, kf - jnp.int32(KH), kf)
    gidx = jnp.where(is_pad, lane, kf_loc * jnp.int32(N) + ii)

    cin = [carry[0], carry[1], carry[2], carry[3]]
    c1r = (oi >= MH).astype(jnp.int32)
    hr = (kf >= KH).astype(jnp.int32)
    clsr = 2 * hr + c1r
    pad_rank = eidx - jnp.int32(E)
    sreal = [jnp.sum(jnp.where(is_pad, 0,
                               (clsr == k).astype(jnp.int32)))
             for k in range(4)]
    padn = []
    tacc = 0
    for k in range(3):
        pk = (-(cin[k] + sreal[k])) % jnp.int32(EW2)
        padn.append(tacc + pk)
        tacc = tacc + pk
    clspad = (jnp.where(pad_rank < padn[0], 0,
              jnp.where(pad_rank < padn[1], 1,
                        jnp.where(pad_rank < padn[2], 2, 3))))
    cls = jnp.where(is_pad, clspad, clsr)

    trash = jnp.int32(MH) + (lane & 63)
    li = jnp.where(is_pad, trash,
                   jnp.where(c1r == 1, oi - jnp.int32(MH), oi))

    iflat = row * EW + lane
    pos = []
    scls = []
    for k in range(4):
        bf = (cls == k).astype(jnp.float32)
        excl = jnp.dot(bf, _tri_lanes(EW),
                       preferred_element_type=jnp.float32)
        rs = jnp.dot(bf, jnp.ones((EW, 1), jnp.float32),
                     preferred_element_type=jnp.float32)
        rp = jnp.dot(_tri_rows(BW), rs, preferred_element_type=jnp.float32)
        pos.append((rp + excl).astype(jnp.int32))
        scls.append(jnp.sum((cls == k).astype(jnp.int32)))
    d0 = cin[0] + pos[0]
    d1 = jnp.int32(EP - 1) - (cin[1] + pos[1])
    d2 = jnp.int32(EP) + cin[2] + pos[2]
    d3 = jnp.int32(2 * EP - 1) - (cin[3] + pos[3])
    dst = jnp.where(cls == 0, d0,
                    jnp.where(cls == 1, d1, jnp.where(cls == 2, d2, d3)))

    p_ref[0] = gidx
    p_ref[1] = li
    p_ref[2] = dst

    for k in range(4):
        carry[k] = cin[k] + scls[k]

    @pl.when(i == (NWINP // BW) - 1)
    def _():
        l2 = lax.broadcasted_iota(jnp.int32, (1, C), 1)
        out = jnp.zeros((1, C), jnp.int32)
        for k in range(4):
            out = jnp.where(l2 == k,
                            (cin[k] + scls[k]) // jnp.int32(EW2), out)
        cnt_ref[...] = out


def _stage_p(in_idx, koff, out_idx):
    ii = jnp.pad(in_idx, (0, EP - E)).reshape(NWINP, EW)
    kf = jnp.pad(koff, (0, EP - E)).reshape(NWINP, EW)
    oi = jnp.pad(out_idx, (0, EP - E)).reshape(NWINP, EW)
    blk = pl.BlockSpec((BW, EW), lambda i: (i, 0))
    return pl.pallas_call(
        _stage_p_body,
        grid=(NWINP // BW,),
        in_specs=[blk, blk, blk],
        out_specs=[
            pl.BlockSpec((3, BW, EW), lambda i: (0, i, 0)),
            pl.BlockSpec((1, C), lambda i: (0, 0)),
        ],
        out_shape=[
            jax.ShapeDtypeStruct((3, NWINP, EW), jnp.int32),
            jax.ShapeDtypeStruct((1, C), jnp.int32),
        ],
        scratch_shapes=[pltpu.SMEM((4,), jnp.int32)],
    )(ii, kf, oi)


def _reorder_kernel(P):
    mesh = plsc.VectorSubcoreMesh(**_SC_MESH)

    @functools.partial(
        pl.kernel,
        out_type=jax.ShapeDtypeStruct((2 * EP, 16), jnp.int32),
        mesh=mesh,
        compiler_params=_sc_params(),
        scratch_types=[
            pltpu.VMEM((NB, EW), jnp.int32),
            pltpu.VMEM((NB, EW), jnp.int32),
            pltpu.VMEM((NB, EW), jnp.int32),
            pltpu.VMEM((NB, EW, 16), jnp.int32),
        ] + [pltpu.SemaphoreType.DMA] * (2 * NB),
    )
    def body(p_hbm, o_hbm, gi_v, li_v, di_v, pe, *sems):
        isem = sems[:NB]
        ssem = sems[NB:]
        c = lax.axis_index("c")
        s = lax.axis_index("s")
        wid = s * 2 + c
        rows16 = lax.iota(jnp.int32, 16)
        z16 = jnp.zeros((16,), jnp.int32)

        @pl.loop(0, NWINP // 32 // NB)
        def _it(u):
            for j in range(NB):
                @pl.when(u > 0)
                def _():
                    pltpu.make_async_copy(pe.at[j], o_hbm.at[di_v.at[j]],
                                          ssem[j]).wait()
                w = wid + (u * NB + j) * 32
                pltpu.async_copy(p_hbm.at[0].at[w], gi_v.at[j], isem[j])
                pltpu.async_copy(p_hbm.at[1].at[w], li_v.at[j], isem[j])
                pltpu.async_copy(p_hbm.at[2].at[w], di_v.at[j], isem[j])
            for j in range(NB):
                w = wid + (u * NB + j) * 32
                pltpu.make_async_copy(p_hbm.at[0].at[w],
                                      gi_v.at[j], isem[j]).wait()
                pltpu.make_async_copy(p_hbm.at[1].at[w],
                                      li_v.at[j], isem[j]).wait()
                pltpu.make_async_copy(p_hbm.at[2].at[w],
                                      di_v.at[j], isem[j]).wait()
                for t in range(EW // 16):
                    sl = pl.ds(t * 16, 16)
                    plsc.store_scatter(pe.at[j], [rows16 + t * 16, z16],
                                       gi_v[j, sl])
                    plsc.store_scatter(pe.at[j], [rows16 + t * 16, z16 + 1],
                                       li_v[j, sl])
                pltpu.async_copy(pe.at[j], o_hbm.at[di_v.at[j]], ssem[j])

        for j in range(NB):
            pltpu.make_async_copy(pe.at[j], o_hbm.at[di_v.at[j]],
                                  ssem[j]).wait()

    return body(P)


def _edge_kernel(Ppart, counts, xw2, sinit, z64, h):
    mesh = plsc.VectorSubcoreMesh(**_SC_MESH)
    TSUB = NWIN2 // 16

    @functools.partial(
        pl.kernel,
        out_type=jax.ShapeDtypeStruct((2, MH, C), jnp.float32),
        mesh=mesh,
        compiler_params=_sc_params(),
        scratch_types=[
            pltpu.VMEM((16,), jnp.int32),
            pltpu.VMEM((NBE, EW2, 16), jnp.int32),
            pltpu.VMEM((NBE, EW2), jnp.int32),
            pltpu.VMEM((NBE, EW2), jnp.int32),
            pltpu.VMEM((NBE, EW2, C), jnp.float32),
            pltpu.VMEM_SHARED((AR, C), jnp.float32),
        ] + [pltpu.SemaphoreType.DMA] * (3 * NBE),
    )
    def body(pp_hbm, cnt_hbm, xw_hbm, si_hbm, z_hbm, out_hbm,
             cnt_v, pb, gi_v, li_v, rows, acc, *sems):
        isem = sems[:NBE]
        gsem = sems[NBE:2 * NBE]
        ssem = sems[2 * NBE:]
        c = lax.axis_index("c")
        s = lax.axis_index("s")
        pltpu.sync_copy(si_hbm.at[c].at[pl.ds(s * OSUB, OSUB)],
                        acc.at[pl.ds(s * OSUB, OSUB)])
        pltpu.sync_copy(z_hbm.at[pl.ds(s * 4, 4)],
                        acc.at[pl.ds(MH + s * 4, 4)])
        pltpu.sync_copy(cnt_hbm.at[0].at[pl.ds(0, 16)], cnt_v)
        lane16 = lax.iota(jnp.int32, 16)
        nwin = jnp.sum(jnp.where(lane16 == 2 * h + c, cnt_v[...], 0))
        plsc.subcore_barrier()
        rows16 = lax.iota(jnp.int32, 16)
        z16 = jnp.zeros((16,), jnp.int32)

        @pl.loop(0, 131)
        def _it(u):
            for j in range(NBE):
                w = s + (u * NBE + j) * 16

                @pl.when(w < nwin)
                def _():
                    @pl.when(u > 0)
                    def _():
                        pltpu.make_async_copy(
                            rows.at[j], acc.at[li_v.at[j]], ssem[j]).wait()
                    base = jnp.int32(h * EP) + jnp.where(
                        c == 0, w * EW2, jnp.int32(EP) - (w + 1) * EW2)
                    pltpu.async_copy(pp_hbm.at[pl.ds(base, EW2)], pb.at[j],
                                     isem[j])
            for j in range(NBE):
                w = s + (u * NBE + j) * 16

                @pl.when(w < nwin)
                def _():
                    pltpu.make_async_copy(pp_hbm.at[pl.ds(0, EW2)], pb.at[j],
                                          isem[j]).wait()
                    for t in range(EW2 // 16):
                        sl = pl.ds(t * 16, 16)
                        gi_v[j, sl] = plsc.load_gather(
                            pb.at[j], [rows16 + t * 16, z16])
                        li_v[j, sl] = plsc.load_gather(
                            pb.at[j], [rows16 + t * 16, z16 + 1])
                    pltpu.async_copy(xw_hbm.at[gi_v.at[j]], rows.at[j],
                                     gsem[j])
            for j in range(NBE):
                w = s + (u * NBE + j) * 16

                @pl.when(w < nwin)
                def _():
                    pltpu.make_async_copy(xw_hbm.at[gi_v.at[j]], rows.at[j],
                                          gsem[j]).wait()
                    pltpu.async_copy(rows.at[j], acc.at[li_v.at[j]], ssem[j],
                                     add=True)

        for j in range(NBE):
            @pl.when((s + j * 16) < nwin)
            def _():
                pltpu.make_async_copy(rows.at[j], acc.at[li_v.at[j]],
                                      ssem[j]).wait()

        plsc.subcore_barrier()
        pltpu.sync_copy(acc.at[pl.ds(s * OSUB, OSUB)],
                        out_hbm.at[c].at[pl.ds(s * OSUB, OSUB)])

    return body(Ppart, counts, xw2, sinit, z64)


def _ds_kernel(ds_idx, x):
    mesh = plsc.VectorSubcoreMesh(**_SC_MESH)

    @functools.partial(
        pl.kernel,
        out_type=jax.ShapeDtypeStruct((M, C), jnp.float32),
        mesh=mesh,
        scratch_types=[
            pltpu.VMEM((DW,), jnp.int32),
            pltpu.VMEM((DW, C), jnp.float32),
        ],
    )
    def body(di_hbm, x_hbm, out_hbm, di_v, rows_v):
        c = lax.axis_index("c")
        s = lax.axis_index("s")
        wid = s * 2 + c

        @pl.loop(0, 20)
        def _win(t):
            w = wid + t * 32

            @pl.when(w < NDWIN)
            def _():
                base = w * DW
                pltpu.sync_copy(di_hbm.at[pl.ds(base, DW)], di_v)
                pltpu.sync_copy(x_hbm.at[di_v], rows_v)
                pltpu.sync_copy(rows_v, out_hbm.at[pl.ds(base, DW)])

    return body(ds_idx, x)


def _stage_c_body(s_ref, dpre_ref, w3_ref, g2_ref, b2_ref, g3_ref, b3_ref,
                  wd_ref, gd_ref, bd_ref, out_ref):
    t = _gn_tc(s_ref[...], g2_ref[...], b2_ref[...], G)
    u = jnp.dot(t, w3_ref[...], preferred_element_type=jnp.float32)
    u = _gn_tc(u, g3_ref[...], b3_ref[...], G)
    d = jnp.dot(dpre_ref[...], wd_ref[...], preferred_element_type=jnp.float32)
    d = _gn_tc(d, gd_ref[...], bd_ref[...], G)
    out_ref[...] = u + d


def _stage_c(S, dpre, W3, g2, b2, g3, b3, Wd, gd, bd):
    S = S.reshape(2 * MH, C)
    vec = pl.BlockSpec((1, C), lambda i: (0, 0))
    return pl.pallas_call(
        _stage_c_body,
        grid=(M // BM,),
        in_specs=[
            pl.BlockSpec((BM, C), lambda i: (i, 0)),
            pl.BlockSpec((BM, C), lambda i: (i, 0)),
            pl.BlockSpec((C, C), lambda i: (0, 0)),
            vec, vec, vec, vec,
            pl.BlockSpec((C, C), lambda i: (0, 0)),
            vec, vec,
        ],
        out_specs=pl.BlockSpec((BM, C), lambda i: (i, 0)),
        out_shape=jax.ShapeDtypeStruct((M, C), jnp.float32),
    )(S, dpre, W3, g2.reshape(1, C), b2.reshape(1, C), g3.reshape(1, C),
      b3.reshape(1, C), Wd, gd.reshape(1, C), bd.reshape(1, C))


def kernel(x, W1, g1, b1, W2, g2, b2, W3, g3, b3, Wd, gd, bd,
           in_idx, out_idx, koff, ds_idx):
    in_idx = in_idx.astype(jnp.int32)
    out_idx = out_idx.astype(jnp.int32)
    koff = koff.astype(jnp.int32)
    ds_idx = ds_idx.astype(jnp.int32)

    P, counts = _stage_p(in_idx, koff, out_idx)
    Ppart = _reorder_kernel(P)
    dpre = _ds_kernel(ds_idx, x)
    xw0 = _stage_a(x, W1, g1, b1, W2[:KH]).reshape(KH * N, C)
    zeros_init = jnp.zeros((2, MH, C), jnp.float32)
    z64 = jnp.zeros((64, C), jnp.float32)
    S0 = _edge_kernel(Ppart, counts, xw0, zeros_init, z64, 0)
    xw1 = _stage_a(x, W1, g1, b1, W2[KH:]).reshape((K - KH) * N, C)
    S = _edge_kernel(Ppart, counts, xw1, S0, z64, 1)
    return _stage_c(S, dpre, W3, g2, b2, g3, b3, Wd, gd, bd)


# speedup vs baseline: 1.0777x; 1.0777x over previous
import dataclasses
import functools

import jax
import jax.numpy as jnp
from jax import lax
from jax.experimental import pallas as pl
from jax.experimental.pallas import tpu as pltpu
from jax.experimental.pallas import tpu_sc as plsc

N = 50000
M = 25000
C = 128
E = 400000
K = 27
G = 8
KN = K * N

BN = 400
BM = 1000
EW = 128
NWINP = 3136
EP = NWINP * EW
BW = 112
EB = BW * EW
EW2 = 64
NWIN2 = EP // EW2
TSUB = NWIN2 // 16
MH = 12800
AR = MH + 64
ASUB = AR // 16
OSUB = MH // 16
DW = 40
NDWIN = M // DW
NB = 2
NBE = 3
_EPS = 1e-5

_SC_MESH = dict(core_axis_name="c", subcore_axis_name="s")


def _sc_params():
    cp = pltpu.CompilerParams()
    return dataclasses.replace(cp, needs_layout_passes=False,
                               use_tc_tiling_on_sc=False)


def _gn_tc(h, gamma, beta, groups):
    n, c = h.shape
    cs = c // groups
    gi = lax.broadcasted_iota(jnp.int32, (c, groups), 0) // cs
    gj = lax.broadcasted_iota(jnp.int32, (c, groups), 1)
    Gm = (gi == gj).astype(jnp.float32)
    ti = lax.broadcasted_iota(jnp.int32, (groups, c), 0)
    tj = lax.broadcasted_iota(jnp.int32, (groups, c), 1) // cs
    GmT = (ti == tj).astype(jnp.float32)
    inv_cs = 1.0 / cs
    mu = jnp.dot(h, Gm, preferred_element_type=jnp.float32) * inv_cs
    ex2 = jnp.dot(h * h, Gm, preferred_element_type=jnp.float32) * inv_cs
    var = ex2 - mu * mu
    rstd = lax.rsqrt(var + _EPS)
    mub = jnp.dot(mu, GmT, preferred_element_type=jnp.float32)
    rstdb = jnp.dot(rstd, GmT, preferred_element_type=jnp.float32)
    return (h - mub) * rstdb * gamma + beta


def _stage_a_body(x_ref, w1_ref, g1_ref, b1_ref, w2_ref, xw_ref):
    h = jnp.dot(x_ref[...], w1_ref[...], preferred_element_type=jnp.float32)
    h = _gn_tc(h, g1_ref[...], b1_ref[...], G)
    h = jnp.where(h >= 0, h, 0.01 * h)
    for k in range(K):
        xw_ref[k] = jnp.dot(h, w2_ref[k], preferred_element_type=jnp.float32)


def _stage_a(x, W1, g1, b1, W2):
    return pl.pallas_call(
        _stage_a_body,
        grid=(N // BN,),
        in_specs=[
            pl.BlockSpec((BN, C), lambda i: (i, 0)),
            pl.BlockSpec((C, C), lambda i: (0, 0)),
            pl.BlockSpec((1, C), lambda i: (0, 0)),
            pl.BlockSpec((1, C), lambda i: (0, 0)),
            pl.BlockSpec((K, C, C), lambda i: (0, 0, 0)),
        ],
        out_specs=pl.BlockSpec((K, BN, C), lambda i: (0, i, 0)),
        out_shape=jax.ShapeDtypeStruct((K, N, C), jnp.float32),
    )(x, W1, g1.reshape(1, C), b1.reshape(1, C), W2)


def _tri_lanes(n):
    a = lax.broadcasted_iota(jnp.int32, (n, n), 0)
    b = lax.broadcasted_iota(jnp.int32, (n, n), 1)
    return (a < b).astype(jnp.float32)


def _tri_rows(n):
    a = lax.broadcasted_iota(jnp.int32, (n, n), 0)
    b = lax.broadcasted_iota(jnp.int32, (n, n), 1)
    return (b < a).astype(jnp.float32)


def _stage_p_body(ii_ref, kf_ref, oi_ref, p_ref, cnt_ref, carry):
    i = pl.program_id(0)

    @pl.when(i == 0)
    def _():
        carry[0] = 0
        carry[1] = 0

    ii = ii_ref[...]
    kf = kf_ref[...]
    oi = oi_ref[...]
    row = lax.broadcasted_iota(jnp.int32, (BW, EW), 0)
    lane = lax.broadcasted_iota(jnp.int32, (BW, EW), 1)
    eidx = i * EB + row * EW + lane
    is_pad = eidx >= E
    gidx = jnp.where(is_pad, lane, kf * jnp.int32(N) + ii)

    c0 = carry[0]
    c1 = carry[1]
    b1r = (oi >= MH).astype(jnp.int32)
    s0_real = jnp.sum(jnp.where(is_pad, 0, 1 - b1r))
    n0_real_total = c0 + s0_real
    padn0 = (-n0_real_total) % jnp.int32(EW)
    pad_rank = eidx - jnp.int32(E)
    b1pad = (pad_rank >= padn0).astype(jnp.int32)
    b1i = jnp.where(is_pad, b1pad, b1r)

    trash = jnp.int32(MH) + (lane & 63)
    li = jnp.where(is_pad, trash,
                   jnp.where(b1i == 1, oi - jnp.int32(MH), oi))

    b1f = b1i.astype(jnp.float32)
    excl1 = jnp.dot(b1f, _tri_lanes(EW), preferred_element_type=jnp.float32)
    rs1 = jnp.dot(b1f, jnp.ones((EW, 1), jnp.float32),
                  preferred_element_type=jnp.float32)
    rp1 = jnp.dot(_tri_rows(BW), rs1, preferred_element_type=jnp.float32)
    pos1 = (rp1 + excl1).astype(jnp.int32)
    iflat = row * EW + lane
    pos0 = iflat - pos1
    dst = jnp.where(b1i == 1, jnp.int32(EP - 1) - (c1 + pos1), c0 + pos0)

    p_ref[0] = gidx
    p_ref[1] = li
    p_ref[2] = dst

    s1 = jnp.sum(b1i)
    s0 = jnp.int32(EB) - s1
    carry[0] = c0 + s0
    carry[1] = c1 + s1

    @pl.when(i == (NWINP // BW) - 1)
    def _():
        nwin0 = (c0 + s0) // jnp.int32(EW2)
        l2 = lax.broadcasted_iota(jnp.int32, (1, C), 1)
        cnt_ref[...] = jnp.where(
            l2 == 0, nwin0,
            jnp.where(l2 == 1, jnp.int32(NWIN2) - nwin0, 0))


def _stage_p(in_idx, koff, out_idx):
    ii = jnp.pad(in_idx, (0, EP - E)).reshape(NWINP, EW)
    kf = jnp.pad(koff, (0, EP - E)).reshape(NWINP, EW)
    oi = jnp.pad(out_idx, (0, EP - E)).reshape(NWINP, EW)
    blk = pl.BlockSpec((BW, EW), lambda i: (i, 0))
    return pl.pallas_call(
        _stage_p_body,
        grid=(NWINP // BW,),
        in_specs=[blk, blk, blk],
        out_specs=[
            pl.BlockSpec((3, BW, EW), lambda i: (0, i, 0)),
            pl.BlockSpec((1, C), lambda i: (0, 0)),
        ],
        out_shape=[
            jax.ShapeDtypeStruct((3, NWINP, EW), jnp.int32),
            jax.ShapeDtypeStruct((1, C), jnp.int32),
        ],
        scratch_shapes=[pltpu.SMEM((2,), jnp.int32)],
    )(ii, kf, oi)


def _reorder_kernel(P):
    mesh = plsc.VectorSubcoreMesh(**_SC_MESH)

    @functools.partial(
        pl.kernel,
        out_type=jax.ShapeDtypeStruct((EP, 16), jnp.int32),
        mesh=mesh,
        compiler_params=_sc_params(),
        scratch_types=[
            pltpu.VMEM((NB, EW), jnp.int32),
            pltpu.VMEM((NB, EW), jnp.int32),
            pltpu.VMEM((NB, EW), jnp.int32),
            pltpu.VMEM((NB, EW, 16), jnp.int32),
        ] + [pltpu.SemaphoreType.DMA] * (2 * NB),
    )
    def body(p_hbm, o_hbm, gi_v, li_v, di_v, pe, *sems):
        isem = sems[:NB]
        ssem = sems[NB:]
        c = lax.axis_index("c")
        s = lax.axis_index("s")
        wid = s * 2 + c
        rows16 = lax.iota(jnp.int32, 16)
        z16 = jnp.zeros((16,), jnp.int32)

        @pl.loop(0, NWINP // 32 // NB)
        def _it(u):
            for j in range(NB):
                @pl.when(u > 0)
                def _():
                    pltpu.make_async_copy(pe.at[j], o_hbm.at[di_v.at[j]],
                                          ssem[j]).wait()
                w = wid + (u * NB + j) * 32
                pltpu.async_copy(p_hbm.at[0].at[w], gi_v.at[j], isem[j])
                pltpu.async_copy(p_hbm.at[1].at[w], li_v.at[j], isem[j])
                pltpu.async_copy(p_hbm.at[2].at[w], di_v.at[j], isem[j])
            for j in range(NB):
                w = wid + (u * NB + j) * 32
                pltpu.make_async_copy(p_hbm.at[0].at[w],
                                      gi_v.at[j], isem[j]).wait()
                pltpu.make_async_copy(p_hbm.at[1].at[w],
                                      li_v.at[j], isem[j]).wait()
                pltpu.make_async_copy(p_hbm.at[2].at[w],
                                      di_v.at[j], isem[j]).wait()
                for t in range(EW // 16):
                    sl = pl.ds(t * 16, 16)
                    plsc.store_scatter(pe.at[j], [rows16 + t * 16, z16],
                                       gi_v[j, sl])
                    plsc.store_scatter(pe.at[j], [rows16 + t * 16, z16 + 1],
                                       li_v[j, sl])
                pltpu.async_copy(pe.at[j], o_hbm.at[di_v.at[j]], ssem[j])

        for j in range(NB):
            pltpu.make_async_copy(pe.at[j], o_hbm.at[di_v.at[j]],
                                  ssem[j]).wait()

    return body(P)


def _edge_kernel(Ppart, counts, xw2, zeros_init):
    mesh = plsc.VectorSubcoreMesh(**_SC_MESH)
    TSUB = NWIN2 // 16

    @functools.partial(
        pl.kernel,
        out_type=jax.ShapeDtypeStruct((2, MH, C), jnp.float32),
        mesh=mesh,
        compiler_params=_sc_params(),
        scratch_types=[
            pltpu.VMEM((16,), jnp.int32),
            pltpu.VMEM((NBE, EW2, 16), jnp.int32),
            pltpu.VMEM((NBE, EW2), jnp.int32),
            pltpu.VMEM((NBE, EW2), jnp.int32),
            pltpu.VMEM((NBE, EW2, C), jnp.float32),
            pltpu.VMEM_SHARED((AR, C), jnp.float32),
        ] + [pltpu.SemaphoreType.DMA] * (3 * NBE),
    )
    def body(pp_hbm, cnt_hbm, xw_hbm, z_hbm, out_hbm,
             cnt_v, pb, gi_v, li_v, rows, acc, *sems):
        isem = sems[:NBE]
        gsem = sems[NBE:2 * NBE]
        ssem = sems[2 * NBE:]
        c = lax.axis_index("c")
        s = lax.axis_index("s")
        pltpu.sync_copy(z_hbm, acc.at[pl.ds(s * ASUB, ASUB)])
        pltpu.sync_copy(cnt_hbm.at[0].at[pl.ds(0, 16)], cnt_v)
        lane16 = lax.iota(jnp.int32, 16)
        nwin = jnp.sum(jnp.where(lane16 == c, cnt_v[...], 0))
        plsc.subcore_barrier()
        rows16 = lax.iota(jnp.int32, 16)
        z16 = jnp.zeros((16,), jnp.int32)

        @pl.loop(0, 131)
        def _it(u):
            for j in range(NBE):
                w = s + (u * NBE + j) * 16

                @pl.when(w < nwin)
                def _():
                    @pl.when(u > 0)
                    def _():
                        pltpu.make_async_copy(
                            rows.at[j], acc.at[li_v.at[j]], ssem[j]).wait()
                    base = jnp.where(c == 0, w * EW2,
                                     jnp.int32(EP) - (w + 1) * EW2)
                    pltpu.async_copy(pp_hbm.at[pl.ds(base, EW2)], pb.at[j],
                                     isem[j])
            for j in range(NBE):
                w = s + (u * NBE + j) * 16

                @pl.when(w < nwin)
                def _():
                    pltpu.make_async_copy(pp_hbm.at[pl.ds(0, EW2)], pb.at[j],
                                          isem[j]).wait()
                    for t in range(EW2 // 16):
                        sl = pl.ds(t * 16, 16)
                        gi_v[j, sl] = plsc.load_gather(
                            pb.at[j], [rows16 + t * 16, z16])
                        li_v[j, sl] = plsc.load_gather(
                            pb.at[j], [rows16 + t * 16, z16 + 1])
                    pltpu.async_copy(xw_hbm.at[gi_v.at[j]], rows.at[j],
                                     gsem[j])
            for j in range(NBE):
                w = s + (u * NBE + j) * 16

                @pl.when(w < nwin)
                def _():
                    pltpu.make_async_copy(xw_hbm.at[gi_v.at[j]], rows.at[j],
                                          gsem[j]).wait()
                    pltpu.async_copy(rows.at[j], acc.at[li_v.at[j]], ssem[j],
                                     add=True)

        for j in range(NBE):
            @pl.when((s + j * 16) < nwin)
            def _():
                pltpu.make_async_copy(rows.at[j], acc.at[li_v.at[j]],
                                      ssem[j]).wait()

        plsc.subcore_barrier()
        pltpu.sync_copy(acc.at[pl.ds(s * OSUB, OSUB)],
                        out_hbm.at[c].at[pl.ds(s * OSUB, OSUB)])

    return body(Ppart, counts, xw2, zeros_init)


def _ds_kernel(ds_idx, x):
    mesh = plsc.VectorSubcoreMesh(**_SC_MESH)

    @functools.partial(
        pl.kernel,
        out_type=jax.ShapeDtypeStruct((M, C), jnp.float32),
        mesh=mesh,
        scratch_types=[
            pltpu.VMEM((DW,), jnp.int32),
            pltpu.VMEM((DW, C), jnp.float32),
        ],
    )
    def body(di_hbm, x_hbm, out_hbm, di_v, rows_v):
        c = lax.axis_index("c")
        s = lax.axis_index("s")
        wid = s * 2 + c

        @pl.loop(0, 20)
        def _win(t):
            w = wid + t * 32

            @pl.when(w < NDWIN)
            def _():
                base = w * DW
                pltpu.sync_copy(di_hbm.at[pl.ds(base, DW)], di_v)
                pltpu.sync_copy(x_hbm.at[di_v], rows_v)
                pltpu.sync_copy(rows_v, out_hbm.at[pl.ds(base, DW)])

    return body(ds_idx, x)


def _stage_c_body(s_ref, dpre_ref, w3_ref, g2_ref, b2_ref, g3_ref, b3_ref,
                  wd_ref, gd_ref, bd_ref, out_ref):
    t = _gn_tc(s_ref[...], g2_ref[...], b2_ref[...], G)
    u = jnp.dot(t, w3_ref[...], preferred_element_type=jnp.float32)
    u = _gn_tc(u, g3_ref[...], b3_ref[...], G)
    d = jnp.dot(dpre_ref[...], wd_ref[...], preferred_element_type=jnp.float32)
    d = _gn_tc(d, gd_ref[...], bd_ref[...], G)
    out_ref[...] = u + d


def _stage_c(S, dpre, W3, g2, b2, g3, b3, Wd, gd, bd):
    S = S.reshape(2 * MH, C)
    vec = pl.BlockSpec((1, C), lambda i: (0, 0))
    return pl.pallas_call(
        _stage_c_body,
        grid=(M // BM,),
        in_specs=[
            pl.BlockSpec((BM, C), lambda i: (i, 0)),
            pl.BlockSpec((BM, C), lambda i: (i, 0)),
            pl.BlockSpec((C, C), lambda i: (0, 0)),
            vec, vec, vec, vec,
            pl.BlockSpec((C, C), lambda i: (0, 0)),
            vec, vec,
        ],
        out_specs=pl.BlockSpec((BM, C), lambda i: (i, 0)),
        out_shape=jax.ShapeDtypeStruct((M, C), jnp.float32),
    )(S, dpre, W3, g2.reshape(1, C), b2.reshape(1, C), g3.reshape(1, C),
      b3.reshape(1, C), Wd, gd.reshape(1, C), bd.reshape(1, C))


def kernel(x, W1, g1, b1, W2, g2, b2, W3, g3, b3, Wd, gd, bd,
           in_idx, out_idx, koff, ds_idx):
    in_idx = in_idx.astype(jnp.int32)
    out_idx = out_idx.astype(jnp.int32)
    koff = koff.astype(jnp.int32)
    ds_idx = ds_idx.astype(jnp.int32)

    P, counts = _stage_p(in_idx, koff, out_idx)
    Ppart = _reorder_kernel(P)
    dpre = _ds_kernel(ds_idx, x)
    xw = _stage_a(x, W1, g1, b1, W2)
    xw2 = xw.reshape(KN, C)
    zeros_init = jnp.zeros((ASUB, C), jnp.float32)
    S = _edge_kernel(Ppart, counts, xw2, zeros_init)
    return _stage_c(S, dpre, W3, g2, b2, g3, b3, Wd, gd, bd)
